# baseline (speedup 1.0000x reference)
import functools

import jax
import jax.numpy as jnp
from jax.experimental import pallas as pl
from jax.experimental.pallas import tpu as pltpu


def _gcn_body(feature_ref, w1_ref, w2_ref, adj_ref, out_ref, s1_ref, z_ref):
    phase = pl.program_id(0)
    i = pl.program_id(1)
    nb = pl.num_programs(1)

    @pl.when((phase == 0) & (i == 0))
    def _():
        s1_ref[...] = jnp.dot(
            feature_ref[...], w1_ref[...], preferred_element_type=jnp.float32
        )

    @pl.when(phase == 0)
    def _():
        h = jnp.maximum(
            jnp.dot(adj_ref[...], s1_ref[...], preferred_element_type=jnp.float32),
            0.0,
        )
        z_ref[pl.ds(i * adj_ref.shape[0], adj_ref.shape[0]), :] = jnp.dot(
            h, w2_ref[...], preferred_element_type=jnp.float32
        )

    @pl.when(phase == 1)
    def _():
        out_ref[...] = jnp.dot(
            adj_ref[...], z_ref[...], preferred_element_type=jnp.float32
        )


@functools.partial(jax.jit, static_argnames=("block_m",))
def _gcn(feature, adj, W1, W2, block_m=400):
    n, d_in = feature.shape
    d_hid = W1.shape[1]
    d_out = W2.shape[1]

    return pl.pallas_call(
        _gcn_body,
        grid=(2, n // block_m),
        in_specs=[
            pl.BlockSpec((n, d_in), lambda p, i: (0, 0)),
            pl.BlockSpec((d_in, d_hid), lambda p, i: (0, 0)),
            pl.BlockSpec((d_hid, d_out), lambda p, i: (0, 0)),
            pl.BlockSpec((block_m, n), lambda p, i: (i, 0)),
        ],
        out_specs=pl.BlockSpec((block_m, d_out), lambda p, i: (i, 0)),
        out_shape=jax.ShapeDtypeStruct((n, d_out), jnp.float32),
        scratch_shapes=[
            pltpu.VMEM((n, d_hid), jnp.float32),
            pltpu.VMEM((n, d_out), jnp.float32),
        ],
    )(feature, W1, W2, adj)


def kernel(feature, adj, W1, W2):
    return _gcn(feature, adj, W1, W2)
